# Initial kernel scaffold; baseline (speedup 1.0000x reference)
import jax
import jax.numpy as jnp
from jax.experimental import pallas as pl


def kernel(logits, current_tokens):
    raise NotImplementedError("write your pallas kernel here")


# speedup vs baseline: 36.8548x; 36.8548x over previous
import functools

import jax
import jax.numpy as jnp
from jax import lax
from jax.experimental import pallas as pl
from jax.experimental.pallas import tpu as pltpu
from jax.experimental.pallas import tpu_sc as plsc

_TEMP = 0.8
_K = 50
_P = 0.9
_B = 64
_V = 100000
_NCAND = 256
_CANDBUF = 272
_NBINS = 4096
_CHUNK = 10000
_NCHUNK = _V // _CHUNK
_L = 16
_VPC = _CHUNK // _L
_NCORES = 2
_NSUB = 16
_ROWS_PER_W = _B // (_NCORES * _NSUB)

import numpy as np

_MINF = np.float32(-np.inf)
_NEG1E9 = np.float32(-1e9)


def _monotone_u32(v16):
    s = lax.bitcast_convert_type(v16, jnp.int32)
    return jnp.where(s < 0, ~s, s ^ jnp.int32(-(2**31)))


def _stage_a_body(logits_hbm, oval_hbm, oidx_hbm, chunk_v, hist, cval, cidx):
    wid = lax.axis_index("s") * _NCORES + lax.axis_index("c")
    lane = lax.iota(jnp.int32, _L)
    ones = jnp.ones((_L,), jnp.int32)
    zeros_i = jnp.zeros((_L,), jnp.int32)
    minf16 = jnp.full((_L,), _MINF, jnp.float32)
    neg1_16 = jnp.full((_L,), -1, jnp.int32)

    def do_row(r):
        def zed(i, _):
            hist[pl.ds(i * _L, _L)] = zeros_i
            return 0

        lax.fori_loop(0, (_NBINS * _L) // _L, zed, 0)

        def p1_chunk(c, _):
            pltpu.sync_copy(logits_hbm.at[pl.ds(r * _V + c * _CHUNK, _CHUNK)], chunk_v)

            def p1_vec(i, _):
                v = chunk_v[pl.ds(i * _L, _L)]
                bins = lax.shift_right_logical(_monotone_u32(v), 20)
                plsc.addupdate_scatter(hist, [lane * _NBINS + bins], ones)
                return 0

            lax.fori_loop(0, _VPC, p1_vec, 0)
            return 0

        lax.fori_loop(0, _NCHUNK, p1_chunk, 0)

        def tscan(cc, carry):
            cnt, found, bstar = carry
            q = _NBINS // _L - 1 - cc
            acc = hist[pl.ds(q * _L, _L)]
            for ln in range(1, _L):
                acc = acc + hist[pl.ds(ln * _NBINS + q * _L, _L)]
            cs = plsc.cumsum(lax.rev(acc, (0,)))
            hit = (cs + cnt) >= _K
            j = jnp.max(plsc.all_reduce_ffs(hit))
            anyhit = j < _L
            cand_b = q * _L + (_L - 1) - j
            first = jnp.logical_and(anyhit, jnp.logical_not(found))
            bstar = jnp.where(first, cand_b, bstar)
            found = jnp.logical_or(found, anyhit)
            cnt = cnt + jnp.max(plsc.cumsum(acc))
            return cnt, found, bstar

        _, _, bstar = lax.fori_loop(
            0, _NBINS // _L, tscan, (jnp.int32(0), False, jnp.int32(0))
        )

        for i in range(_CANDBUF // _L):
            cval[pl.ds(i * _L, _L)] = minf16
            cidx[pl.ds(i * _L, _L)] = neg1_16

        def p2_chunk(c, off):
            pltpu.sync_copy(logits_hbm.at[pl.ds(r * _V + c * _CHUNK, _CHUNK)], chunk_v)

            def p2_vec(i, off):
                v = chunk_v[pl.ds(i * _L, _L)]
                bins = lax.shift_right_logical(_monotone_u32(v), 20)
                m = bins >= bstar
                offc = jnp.minimum(off, jnp.int32(_NCAND))
                plsc.store_compressed(cval.at[pl.ds(offc, _L)], v, mask=m)
                gidx = c * _CHUNK + i * _L + lane
                plsc.store_compressed(cidx.at[pl.ds(offc, _L)], gidx, mask=m)
                return off + jnp.max(plsc.all_reduce_population_count(m))

            return lax.fori_loop(0, _VPC, p2_vec, off)

        lax.fori_loop(0, _NCHUNK, p2_chunk, jnp.int32(0))

        pltpu.sync_copy(cval.at[pl.ds(0, _NCAND)], oval_hbm.at[pl.ds(r * _NCAND, _NCAND)])
        pltpu.sync_copy(cidx.at[pl.ds(0, _NCAND)], oidx_hbm.at[pl.ds(r * _NCAND, _NCAND)])

    for rr in range(_ROWS_PER_W):
        do_row(wid * _ROWS_PER_W + rr)


def _threefry_gumbel(flat_idx):
    k0 = jnp.uint32(0)
    k1 = jnp.uint32(42)
    k2 = jnp.uint32(0x1BD11BDA) ^ k0 ^ k1
    ks = (k0, k1, k2)
    rots = ((13, 15, 26, 6), (17, 29, 16, 24))
    x0 = jnp.zeros(flat_idx.shape, jnp.uint32) + ks[0]
    x1 = flat_idx.astype(jnp.uint32) + ks[1]
    for i in range(5):
        for r in rots[i % 2]:
            x0 = x0 + x1
            x1 = (x1 << jnp.uint32(r)) | (x1 >> jnp.uint32(32 - r))
            x1 = x1 ^ x0
        x0 = x0 + ks[(i + 1) % 3]
        x1 = x1 + ks[(i + 2) % 3] + jnp.uint32(i + 1)
    bits = x0 ^ x1
    f = lax.bitcast_convert_type(
        (bits >> jnp.uint32(9)) | jnp.uint32(0x3F800000), jnp.float32
    ) - jnp.float32(1.0)
    u = f * jnp.float32(1.0 - 1e-10) + jnp.float32(1e-10)
    u = jnp.maximum(jnp.float32(1e-10), u)
    return -jnp.log(-jnp.log(u))


def _stage_b_body(vals_ref, idx_ref, probs_ref, tok_ref):
    pid = pl.program_id(0)
    rb = vals_ref.shape[0]
    v = vals_ref[...]
    ci = idx_ref[...]
    valid = ci >= 0
    lc = v / jnp.float32(_TEMP)
    lc_j = lc[:, None, :]
    lc_i = lc[:, :, None]
    gt = lc_j > lc_i
    cnt = jnp.sum(gt.astype(jnp.float32), axis=-1)
    keep = jnp.logical_and(valid, cnt < _K)
    lm = jnp.where(keep, lc, _NEG1E9)
    m1 = jnp.max(lm, axis=-1, keepdims=True)
    e1 = jnp.exp(lm - m1)
    p1 = e1 / jnp.sum(e1, axis=-1, keepdims=True)
    before = jnp.logical_or(
        gt,
        jnp.logical_and(
            lc_j == lc_i,
            jnp.logical_and(ci[:, None, :] <= ci[:, :, None], valid[:, None, :]),
        ),
    )
    cum = jnp.sum(p1[:, None, :] * before.astype(jnp.float32), axis=-1)
    remove = (cum - p1) > jnp.float32(_P)
    l2 = jnp.where(jnp.logical_and(keep, jnp.logical_not(remove)), lm, _NEG1E9)
    m2 = jnp.max(l2, axis=-1, keepdims=True)
    e2 = jnp.exp(l2 - m2)
    p2 = e2 / jnp.sum(e2, axis=-1, keepdims=True)
    probs_ref[...] = p2

    rows = pid * rb + lax.broadcasted_iota(jnp.int32, (rb, _NCAND), 0)
    flat = rows * _V + jnp.maximum(ci, 0)
    g = _threefry_gumbel(flat)
    score = jnp.log(p2 + jnp.float32(1e-12)) + g
    score = jnp.where(
        jnp.logical_and(valid, p2 > 0), score, jnp.float32(-1e30)
    )
    best = jnp.max(score, axis=-1, keepdims=True)
    tok = jnp.min(jnp.where(score == best, ci, jnp.int32(_V)), axis=-1, keepdims=True)
    tok_ref[...] = tok


def _stage_c_body(pc_hbm, ci_hbm, out_hbm, rowbuf, pval, pidx):
    wid = lax.axis_index("s") * _NCORES + lax.axis_index("c")
    zeros_f = jnp.zeros((_L,), jnp.float32)

    def zed(i, _):
        rowbuf[pl.ds(i * _L, _L)] = zeros_f
        return 0

    lax.fori_loop(0, _V // _L, zed, 0)

    def do_row(r):
        pltpu.sync_copy(pc_hbm.at[pl.ds(r * _NCAND, _NCAND)], pval)
        pltpu.sync_copy(ci_hbm.at[pl.ds(r * _NCAND, _NCAND)], pidx)
        for i in range(_NCAND // _L):
            idxv = pidx[pl.ds(i * _L, _L)]
            m = idxv >= 0
            plsc.store_scatter(rowbuf, [idxv], pval[pl.ds(i * _L, _L)], mask=m)
        pltpu.sync_copy(rowbuf, out_hbm.at[pl.ds(r * _V, _V)])
        for i in range(_NCAND // _L):
            idxv = pidx[pl.ds(i * _L, _L)]
            m = idxv >= 0
            plsc.store_scatter(rowbuf, [idxv], zeros_f, mask=m)

    for rr in range(_ROWS_PER_W):
        do_row(wid * _ROWS_PER_W + rr)


@functools.lru_cache(maxsize=1)
def _build_sc_stages():
    mesh = plsc.VectorSubcoreMesh(
        core_axis_name="c", subcore_axis_name="s",
        num_cores=_NCORES, num_subcores=_NSUB,
    )
    stage_a = pl.kernel(
        _stage_a_body,
        out_type=(
            jax.ShapeDtypeStruct((_B * _NCAND,), jnp.float32),
            jax.ShapeDtypeStruct((_B * _NCAND,), jnp.int32),
        ),
        mesh=mesh,
        scratch_types=[
            pltpu.VMEM((_CHUNK,), jnp.float32),
            pltpu.VMEM((_NBINS * _L,), jnp.int32),
            pltpu.VMEM((_CANDBUF,), jnp.float32),
            pltpu.VMEM((_CANDBUF,), jnp.int32),
        ],
        name="sampler_stage_a_select",
        compiler_params=pltpu.CompilerParams(needs_layout_passes=False),
    )
    stage_c = pl.kernel(
        _stage_c_body,
        out_type=jax.ShapeDtypeStruct((_B * _V,), jnp.float32),
        mesh=mesh,
        scratch_types=[
            pltpu.VMEM((_V,), jnp.float32),
            pltpu.VMEM((_NCAND,), jnp.float32),
            pltpu.VMEM((_NCAND,), jnp.int32),
        ],
        name="sampler_stage_c_scatter",
        compiler_params=pltpu.CompilerParams(needs_layout_passes=False),
    )
    return stage_a, stage_c


_RB = 8

_stage_b = pl.pallas_call(
    _stage_b_body,
    grid=(_B // _RB,),
    in_specs=[
        pl.BlockSpec((_RB, _NCAND), lambda i: (i, 0)),
        pl.BlockSpec((_RB, _NCAND), lambda i: (i, 0)),
    ],
    out_specs=[
        pl.BlockSpec((_RB, _NCAND), lambda i: (i, 0)),
        pl.BlockSpec((_RB, 1), lambda i: (i, 0)),
    ],
    out_shape=[
        jax.ShapeDtypeStruct((_B, _NCAND), jnp.float32),
        jax.ShapeDtypeStruct((_B, 1), jnp.int32),
    ],
    name="sampler_stage_b_math",
)

def kernel(logits, current_tokens):
    del current_tokens
    _stage_a, _stage_c = _build_sc_stages()
    cand_vals, cand_idx = _stage_a(logits.reshape(_B * _V))
    probs_c, tok = _stage_b(
        cand_vals.reshape(_B, _NCAND), cand_idx.reshape(_B, _NCAND)
    )
    probs = _stage_c(probs_c.reshape(_B * _NCAND), cand_idx)
    return probs.reshape(_B, _V), tok.reshape(_B)


# speedup vs baseline: 37.1787x; 1.0088x over previous
import functools

import jax
import jax.numpy as jnp
from jax import lax
from jax.experimental import pallas as pl
from jax.experimental.pallas import tpu as pltpu
from jax.experimental.pallas import tpu_sc as plsc

_TEMP = 0.8
_K = 50
_P = 0.9
_B = 64
_V = 100000
_NCAND = 256
_CANDBUF = 272
_NBINS = 4096
_CHUNK = 10000
_NCHUNK = _V // _CHUNK
_L = 16
_VPC = _CHUNK // _L
_NCORES = 2
_NSUB = 16
_UNROLL = 25
_ROWS_PER_W = _B // (_NCORES * _NSUB)

import numpy as np

_MINF = np.float32(-np.inf)
_NEG1E9 = np.float32(-1e9)


def _monotone_u32(v16):
    s = lax.bitcast_convert_type(v16, jnp.int32)
    m = lax.shift_right_arithmetic(s, 31)
    return s ^ (m | jnp.int32(-(2**31)))


def _stage_a_body(logits_hbm, oval_hbm, oidx_hbm, chunk_v, hist, cval, cidx):
    wid = lax.axis_index("s") * _NCORES + lax.axis_index("c")
    lane = lax.iota(jnp.int32, _L)
    ones = jnp.ones((_L,), jnp.int32)
    zeros_i = jnp.zeros((_L,), jnp.int32)
    minf16 = jnp.full((_L,), _MINF, jnp.float32)
    neg1_16 = jnp.full((_L,), -1, jnp.int32)

    def do_row(r):
        def zed(i, _):
            hist[pl.ds(i * _L, _L)] = zeros_i
            return 0

        lax.fori_loop(0, (_NBINS * _L) // _L, zed, 0)

        def p1_chunk(c, _):
            pltpu.sync_copy(logits_hbm.at[pl.ds(r * _V + c * _CHUNK, _CHUNK)], chunk_v)

            def p1_vec(i, _):
                for k in range(_UNROLL):
                    v = chunk_v[pl.ds((i * _UNROLL + k) * _L, _L)]
                    bins = lax.shift_right_logical(_monotone_u32(v), 20)
                    plsc.addupdate_scatter(hist, [lane * _NBINS + bins], ones)
                return 0

            lax.fori_loop(0, _VPC // _UNROLL, p1_vec, 0)
            return 0

        lax.fori_loop(0, _NCHUNK, p1_chunk, 0)

        def tscan(cc, carry):
            cnt, found, bstar = carry
            q = _NBINS // _L - 1 - cc
            acc = hist[pl.ds(q * _L, _L)]
            for ln in range(1, _L):
                acc = acc + hist[pl.ds(ln * _NBINS + q * _L, _L)]
            cs = plsc.cumsum(lax.rev(acc, (0,)))
            hit = (cs + cnt) >= _K
            j = jnp.max(plsc.all_reduce_ffs(hit))
            anyhit = j < _L
            cand_b = q * _L + (_L - 1) - j
            first = jnp.logical_and(anyhit, jnp.logical_not(found))
            bstar = jnp.where(first, cand_b, bstar)
            found = jnp.logical_or(found, anyhit)
            cnt = cnt + jnp.max(plsc.cumsum(acc))
            return cnt, found, bstar

        _, _, bstar = lax.fori_loop(
            0, _NBINS // _L, tscan, (jnp.int32(0), False, jnp.int32(0))
        )

        for i in range(_CANDBUF // _L):
            cval[pl.ds(i * _L, _L)] = minf16
            cidx[pl.ds(i * _L, _L)] = neg1_16

        def p2_chunk(c, off):
            pltpu.sync_copy(logits_hbm.at[pl.ds(r * _V + c * _CHUNK, _CHUNK)], chunk_v)

            def p2_vec(i, off):
                for k in range(_UNROLL):
                    v = chunk_v[pl.ds((i * _UNROLL + k) * _L, _L)]
                    bins = lax.shift_right_logical(_monotone_u32(v), 20)
                    m = bins >= bstar
                    offc = jnp.minimum(off, jnp.int32(_NCAND))
                    plsc.store_compressed(cval.at[pl.ds(offc, _L)], v, mask=m)
                    gidx = c * _CHUNK + (i * _UNROLL + k) * _L + lane
                    plsc.store_compressed(cidx.at[pl.ds(offc, _L)], gidx, mask=m)
                    off = off + jnp.max(plsc.all_reduce_population_count(m))
                return off

            return lax.fori_loop(0, _VPC // _UNROLL, p2_vec, off)

        lax.fori_loop(0, _NCHUNK, p2_chunk, jnp.int32(0))

        pltpu.sync_copy(cval.at[pl.ds(0, _NCAND)], oval_hbm.at[pl.ds(r * _NCAND, _NCAND)])
        pltpu.sync_copy(cidx.at[pl.ds(0, _NCAND)], oidx_hbm.at[pl.ds(r * _NCAND, _NCAND)])

    for rr in range(_ROWS_PER_W):
        do_row(wid * _ROWS_PER_W + rr)


def _threefry_gumbel(flat_idx):
    k0 = jnp.uint32(0)
    k1 = jnp.uint32(42)
    k2 = jnp.uint32(0x1BD11BDA) ^ k0 ^ k1
    ks = (k0, k1, k2)
    rots = ((13, 15, 26, 6), (17, 29, 16, 24))
    x0 = jnp.zeros(flat_idx.shape, jnp.uint32) + ks[0]
    x1 = flat_idx.astype(jnp.uint32) + ks[1]
    for i in range(5):
        for r in rots[i % 2]:
            x0 = x0 + x1
            x1 = (x1 << jnp.uint32(r)) | (x1 >> jnp.uint32(32 - r))
            x1 = x1 ^ x0
        x0 = x0 + ks[(i + 1) % 3]
        x1 = x1 + ks[(i + 2) % 3] + jnp.uint32(i + 1)
    bits = x0 ^ x1
    f = lax.bitcast_convert_type(
        (bits >> jnp.uint32(9)) | jnp.uint32(0x3F800000), jnp.float32
    ) - jnp.float32(1.0)
    u = f * jnp.float32(1.0 - 1e-10) + jnp.float32(1e-10)
    u = jnp.maximum(jnp.float32(1e-10), u)
    return -jnp.log(-jnp.log(u))


def _stage_b_body(vals_ref, idx_ref, probs_ref, tok_ref):
    pid = pl.program_id(0)
    rb = vals_ref.shape[0]
    v = vals_ref[...]
    ci = idx_ref[...]
    valid = ci >= 0
    lc = v / jnp.float32(_TEMP)
    lc_j = lc[:, None, :]
    lc_i = lc[:, :, None]
    gt = lc_j > lc_i
    cnt = jnp.sum(gt.astype(jnp.float32), axis=-1)
    keep = jnp.logical_and(valid, cnt < _K)
    lm = jnp.where(keep, lc, _NEG1E9)
    m1 = jnp.max(lm, axis=-1, keepdims=True)
    e1 = jnp.exp(lm - m1)
    p1 = e1 / jnp.sum(e1, axis=-1, keepdims=True)
    before = jnp.logical_or(
        gt,
        jnp.logical_and(
            lc_j == lc_i,
            jnp.logical_and(ci[:, None, :] <= ci[:, :, None], valid[:, None, :]),
        ),
    )
    cum = jnp.sum(p1[:, None, :] * before.astype(jnp.float32), axis=-1)
    remove = (cum - p1) > jnp.float32(_P)
    l2 = jnp.where(jnp.logical_and(keep, jnp.logical_not(remove)), lm, _NEG1E9)
    m2 = jnp.max(l2, axis=-1, keepdims=True)
    e2 = jnp.exp(l2 - m2)
    p2 = e2 / jnp.sum(e2, axis=-1, keepdims=True)
    probs_ref[...] = p2

    rows = pid * rb + lax.broadcasted_iota(jnp.int32, (rb, _NCAND), 0)
    flat = rows * _V + jnp.maximum(ci, 0)
    g = _threefry_gumbel(flat)
    score = jnp.log(p2 + jnp.float32(1e-12)) + g
    score = jnp.where(
        jnp.logical_and(valid, p2 > 0), score, jnp.float32(-1e30)
    )
    best = jnp.max(score, axis=-1, keepdims=True)
    tok = jnp.min(jnp.where(score == best, ci, jnp.int32(_V)), axis=-1, keepdims=True)
    tok_ref[...] = tok


def _stage_c_body(pc_hbm, ci_hbm, out_hbm, rowbuf, pval, pidx):
    wid = lax.axis_index("s") * _NCORES + lax.axis_index("c")
    zeros_f = jnp.zeros((_L,), jnp.float32)

    def zed(i, _):
        for k in range(_UNROLL):
            rowbuf[pl.ds((i * _UNROLL + k) * _L, _L)] = zeros_f
        return 0

    lax.fori_loop(0, _V // (_L * _UNROLL), zed, 0)

    def do_row(r):
        pltpu.sync_copy(pc_hbm.at[pl.ds(r * _NCAND, _NCAND)], pval)
        pltpu.sync_copy(ci_hbm.at[pl.ds(r * _NCAND, _NCAND)], pidx)
        for i in range(_NCAND // _L):
            idxv = pidx[pl.ds(i * _L, _L)]
            m = idxv >= 0
            plsc.store_scatter(rowbuf, [idxv], pval[pl.ds(i * _L, _L)], mask=m)
        pltpu.sync_copy(rowbuf, out_hbm.at[pl.ds(r * _V, _V)])
        for i in range(_NCAND // _L):
            idxv = pidx[pl.ds(i * _L, _L)]
            m = idxv >= 0
            plsc.store_scatter(rowbuf, [idxv], zeros_f, mask=m)

    for rr in range(_ROWS_PER_W):
        do_row(wid * _ROWS_PER_W + rr)


@functools.lru_cache(maxsize=1)
def _build_sc_stages():
    mesh = plsc.VectorSubcoreMesh(
        core_axis_name="c", subcore_axis_name="s",
        num_cores=_NCORES, num_subcores=_NSUB,
    )
    stage_a = pl.kernel(
        _stage_a_body,
        out_type=(
            jax.ShapeDtypeStruct((_B * _NCAND,), jnp.float32),
            jax.ShapeDtypeStruct((_B * _NCAND,), jnp.int32),
        ),
        mesh=mesh,
        scratch_types=[
            pltpu.VMEM((_CHUNK,), jnp.float32),
            pltpu.VMEM((_NBINS * _L,), jnp.int32),
            pltpu.VMEM((_CANDBUF,), jnp.float32),
            pltpu.VMEM((_CANDBUF,), jnp.int32),
        ],
        name="sampler_stage_a_select",
        compiler_params=pltpu.CompilerParams(needs_layout_passes=False),
    )
    stage_c = pl.kernel(
        _stage_c_body,
        out_type=jax.ShapeDtypeStruct((_B * _V,), jnp.float32),
        mesh=mesh,
        scratch_types=[
            pltpu.VMEM((_V,), jnp.float32),
            pltpu.VMEM((_NCAND,), jnp.float32),
            pltpu.VMEM((_NCAND,), jnp.int32),
        ],
        name="sampler_stage_c_scatter",
        compiler_params=pltpu.CompilerParams(needs_layout_passes=False),
    )
    return stage_a, stage_c


_RB = 8

_stage_b = pl.pallas_call(
    _stage_b_body,
    grid=(_B // _RB,),
    in_specs=[
        pl.BlockSpec((_RB, _NCAND), lambda i: (i, 0)),
        pl.BlockSpec((_RB, _NCAND), lambda i: (i, 0)),
    ],
    out_specs=[
        pl.BlockSpec((_RB, _NCAND), lambda i: (i, 0)),
        pl.BlockSpec((_RB, 1), lambda i: (i, 0)),
    ],
    out_shape=[
        jax.ShapeDtypeStruct((_B, _NCAND), jnp.float32),
        jax.ShapeDtypeStruct((_B, 1), jnp.int32),
    ],
    name="sampler_stage_b_math",
)

def kernel(logits, current_tokens):
    del current_tokens
    _stage_a, _stage_c = _build_sc_stages()
    cand_vals, cand_idx = _stage_a(logits.reshape(_B * _V))
    probs_c, tok = _stage_b(
        cand_vals.reshape(_B, _NCAND), cand_idx.reshape(_B, _NCAND)
    )
    probs = _stage_c(probs_c.reshape(_B * _NCAND), cand_idx)
    return probs.reshape(_B, _V), tok.reshape(_B)


# speedup vs baseline: 41.0863x; 1.1051x over previous
import functools

import jax
import jax.numpy as jnp
from jax import lax
from jax.experimental import pallas as pl
from jax.experimental.pallas import tpu as pltpu
from jax.experimental.pallas import tpu_sc as plsc

_TEMP = 0.8
_K = 50
_P = 0.9
_B = 64
_V = 100000
_NCAND = 256
_CANDBUF = 272
_SLOTS = 64
_NBINS = 4096
_CHUNK = 10000
_NCHUNK = _V // _CHUNK
_L = 16
_VPC = _CHUNK // _L
_NCORES = 2
_NSUB = 16
_UNROLL = 25
_ROWS_PER_W = _B // (_NCORES * _NSUB)

import numpy as np

_MINF = np.float32(-np.inf)
_NEG1E9 = np.float32(-1e9)


def _lane(x16, i):
    return jnp.squeeze(lax.slice_in_dim(x16, i, i + 1, axis=0))


def _monotone_u32(v16):
    s = lax.bitcast_convert_type(v16, jnp.int32)
    m = lax.shift_right_arithmetic(s, 31)
    return s ^ (m | jnp.int32(-(2**31)))


def _stage_a_body(logits_hbm, oval_hbm, oidx_hbm, chunk_v, hist, cval, cidx, lbval, lbidx):
    wid = lax.axis_index("s") * _NCORES + lax.axis_index("c")
    lane = lax.iota(jnp.int32, _L)
    ones = jnp.ones((_L,), jnp.int32)
    zeros_i = jnp.zeros((_L,), jnp.int32)
    minf16 = jnp.full((_L,), _MINF, jnp.float32)
    neg1_16 = jnp.full((_L,), -1, jnp.int32)

    def do_row(r):
        def zed(i, _):
            hist[pl.ds(i * _L, _L)] = zeros_i
            return 0

        lax.fori_loop(0, (_NBINS * _L) // _L, zed, 0)

        def p1_chunk(c, _):
            pltpu.sync_copy(logits_hbm.at[pl.ds(r * _V + c * _CHUNK, _CHUNK)], chunk_v)

            def p1_vec(i, _):
                for k in range(_UNROLL):
                    v = chunk_v[pl.ds((i * _UNROLL + k) * _L, _L)]
                    bins = lax.shift_right_logical(_monotone_u32(v), 20)
                    plsc.addupdate_scatter(hist, [lane * _NBINS + bins], ones)
                return 0

            lax.fori_loop(0, _VPC // _UNROLL, p1_vec, 0)
            return 0

        lax.fori_loop(0, _NCHUNK, p1_chunk, 0)

        def tscan(cc, carry):
            cnt, found, bstar = carry
            q = _NBINS // _L - 1 - cc
            acc = hist[pl.ds(q * _L, _L)]
            for ln in range(1, _L):
                acc = acc + hist[pl.ds(ln * _NBINS + q * _L, _L)]
            cs = plsc.cumsum(lax.rev(acc, (0,)))
            hit = (cs + cnt) >= _K
            j = _lane(plsc.all_reduce_ffs(hit), 0)
            anyhit = j < _L
            cand_b = q * _L + (_L - 1) - j
            first = jnp.logical_and(anyhit, jnp.logical_not(found))
            bstar = jnp.where(first, cand_b, bstar)
            found = jnp.logical_or(found, anyhit)
            cnt = cnt + _lane(cs, _L - 1)
            return cnt, found, bstar

        _, _, bstar = lax.fori_loop(
            0, _NBINS // _L, tscan, (jnp.int32(0), False, jnp.int32(0))
        )

        for i in range(_CANDBUF // _L):
            cval[pl.ds(i * _L, _L)] = minf16
            cidx[pl.ds(i * _L, _L)] = neg1_16

        def p2_chunk(c, percnt):
            pltpu.sync_copy(logits_hbm.at[pl.ds(r * _V + c * _CHUNK, _CHUNK)], chunk_v)

            def p2_vec(i, percnt):
                for k in range(_UNROLL):
                    v = chunk_v[pl.ds((i * _UNROLL + k) * _L, _L)]
                    bins = lax.shift_right_logical(_monotone_u32(v), 20)
                    m = bins >= bstar
                    slot = jnp.minimum(percnt, jnp.int32(_SLOTS - 1)) * _L + lane
                    plsc.store_scatter(lbval, [slot], v, mask=m)
                    gidx = c * _CHUNK + (i * _UNROLL + k) * _L + lane
                    plsc.store_scatter(lbidx, [slot], gidx, mask=m)
                    percnt = percnt + m.astype(jnp.int32)
                return percnt

            return lax.fori_loop(0, _VPC // _UNROLL, p2_vec, percnt)

        percnt = lax.fori_loop(0, _NCHUNK, p2_chunk, jnp.zeros((_L,), jnp.int32))

        def compact(sl, off):
            m = percnt > sl
            offc = jnp.minimum(off, jnp.int32(_NCAND))
            plsc.store_compressed(cval.at[pl.ds(offc, _L)], lbval[pl.ds(sl * _L, _L)], mask=m)
            plsc.store_compressed(cidx.at[pl.ds(offc, _L)], lbidx[pl.ds(sl * _L, _L)], mask=m)
            return off + _lane(plsc.all_reduce_population_count(m), 0)

        lax.fori_loop(0, _SLOTS, compact, jnp.int32(0))

        pltpu.sync_copy(cval.at[pl.ds(0, _NCAND)], oval_hbm.at[pl.ds(r * _NCAND, _NCAND)])
        pltpu.sync_copy(cidx.at[pl.ds(0, _NCAND)], oidx_hbm.at[pl.ds(r * _NCAND, _NCAND)])

    for rr in range(_ROWS_PER_W):
        do_row(wid * _ROWS_PER_W + rr)


def _threefry_gumbel(flat_idx):
    k0 = jnp.uint32(0)
    k1 = jnp.uint32(42)
    k2 = jnp.uint32(0x1BD11BDA) ^ k0 ^ k1
    ks = (k0, k1, k2)
    rots = ((13, 15, 26, 6), (17, 29, 16, 24))
    x0 = jnp.zeros(flat_idx.shape, jnp.uint32) + ks[0]
    x1 = flat_idx.astype(jnp.uint32) + ks[1]
    for i in range(5):
        for r in rots[i % 2]:
            x0 = x0 + x1
            x1 = (x1 << jnp.uint32(r)) | (x1 >> jnp.uint32(32 - r))
            x1 = x1 ^ x0
        x0 = x0 + ks[(i + 1) % 3]
        x1 = x1 + ks[(i + 2) % 3] + jnp.uint32(i + 1)
    bits = x0 ^ x1
    f = lax.bitcast_convert_type(
        (bits >> jnp.uint32(9)) | jnp.uint32(0x3F800000), jnp.float32
    ) - jnp.float32(1.0)
    u = f * jnp.float32(1.0 - 1e-10) + jnp.float32(1e-10)
    u = jnp.maximum(jnp.float32(1e-10), u)
    return -jnp.log(-jnp.log(u))


def _stage_b_body(vals_ref, idx_ref, probs_ref, tok_ref):
    pid = pl.program_id(0)
    rb = vals_ref.shape[0]
    v = vals_ref[...]
    ci = idx_ref[...]
    valid = ci >= 0
    lc = v / jnp.float32(_TEMP)
    lc_j = lc[:, None, :]
    lc_i = lc[:, :, None]
    gt = lc_j > lc_i
    cnt = jnp.sum(gt.astype(jnp.float32), axis=-1)
    keep = jnp.logical_and(valid, cnt < _K)
    lm = jnp.where(keep, lc, _NEG1E9)
    m1 = jnp.max(lm, axis=-1, keepdims=True)
    e1 = jnp.exp(lm - m1)
    p1 = e1 / jnp.sum(e1, axis=-1, keepdims=True)
    before = jnp.logical_or(
        gt,
        jnp.logical_and(
            lc_j == lc_i,
            jnp.logical_and(ci[:, None, :] <= ci[:, :, None], valid[:, None, :]),
        ),
    )
    cum = jnp.sum(p1[:, None, :] * before.astype(jnp.float32), axis=-1)
    remove = (cum - p1) > jnp.float32(_P)
    l2 = jnp.where(jnp.logical_and(keep, jnp.logical_not(remove)), lm, _NEG1E9)
    m2 = jnp.max(l2, axis=-1, keepdims=True)
    e2 = jnp.exp(l2 - m2)
    p2 = e2 / jnp.sum(e2, axis=-1, keepdims=True)
    probs_ref[...] = p2

    rows = pid * rb + lax.broadcasted_iota(jnp.int32, (rb, _NCAND), 0)
    flat = rows * _V + jnp.maximum(ci, 0)
    g = _threefry_gumbel(flat)
    score = jnp.log(p2 + jnp.float32(1e-12)) + g
    score = jnp.where(
        jnp.logical_and(valid, p2 > 0), score, jnp.float32(-1e30)
    )
    best = jnp.max(score, axis=-1, keepdims=True)
    tok = jnp.min(jnp.where(score == best, ci, jnp.int32(_V)), axis=-1, keepdims=True)
    tok_ref[...] = tok


def _stage_c_body(pc_hbm, ci_hbm, out_hbm, rowbuf, pval, pidx):
    wid = lax.axis_index("s") * _NCORES + lax.axis_index("c")
    zeros_f = jnp.zeros((_L,), jnp.float32)

    def zed(i, _):
        for k in range(_UNROLL):
            rowbuf[pl.ds((i * _UNROLL + k) * _L, _L)] = zeros_f
        return 0

    lax.fori_loop(0, _V // (_L * _UNROLL), zed, 0)

    def do_row(r):
        pltpu.sync_copy(pc_hbm.at[pl.ds(r * _NCAND, _NCAND)], pval)
        pltpu.sync_copy(ci_hbm.at[pl.ds(r * _NCAND, _NCAND)], pidx)
        for i in range(_NCAND // _L):
            idxv = pidx[pl.ds(i * _L, _L)]
            m = idxv >= 0
            plsc.store_scatter(rowbuf, [idxv], pval[pl.ds(i * _L, _L)], mask=m)
        pltpu.sync_copy(rowbuf, out_hbm.at[pl.ds(r * _V, _V)])
        for i in range(_NCAND // _L):
            idxv = pidx[pl.ds(i * _L, _L)]
            m = idxv >= 0
            plsc.store_scatter(rowbuf, [idxv], zeros_f, mask=m)

    for rr in range(_ROWS_PER_W):
        do_row(wid * _ROWS_PER_W + rr)


@functools.lru_cache(maxsize=1)
def _build_sc_stages():
    mesh = plsc.VectorSubcoreMesh(
        core_axis_name="c", subcore_axis_name="s",
        num_cores=_NCORES, num_subcores=_NSUB,
    )
    stage_a = pl.kernel(
        _stage_a_body,
        out_type=(
            jax.ShapeDtypeStruct((_B * _NCAND,), jnp.float32),
            jax.ShapeDtypeStruct((_B * _NCAND,), jnp.int32),
        ),
        mesh=mesh,
        scratch_types=[
            pltpu.VMEM((_CHUNK,), jnp.float32),
            pltpu.VMEM((_NBINS * _L,), jnp.int32),
            pltpu.VMEM((_CANDBUF,), jnp.float32),
            pltpu.VMEM((_CANDBUF,), jnp.int32),
            pltpu.VMEM((_SLOTS * _L,), jnp.float32),
            pltpu.VMEM((_SLOTS * _L,), jnp.int32),
        ],
        name="sampler_stage_a_select",
        compiler_params=pltpu.CompilerParams(needs_layout_passes=False),
    )
    stage_c = pl.kernel(
        _stage_c_body,
        out_type=jax.ShapeDtypeStruct((_B * _V,), jnp.float32),
        mesh=mesh,
        scratch_types=[
            pltpu.VMEM((_V,), jnp.float32),
            pltpu.VMEM((_NCAND,), jnp.float32),
            pltpu.VMEM((_NCAND,), jnp.int32),
        ],
        name="sampler_stage_c_scatter",
        compiler_params=pltpu.CompilerParams(needs_layout_passes=False),
    )
    return stage_a, stage_c


_RB = 8

_stage_b = pl.pallas_call(
    _stage_b_body,
    grid=(_B // _RB,),
    in_specs=[
        pl.BlockSpec((_RB, _NCAND), lambda i: (i, 0)),
        pl.BlockSpec((_RB, _NCAND), lambda i: (i, 0)),
    ],
    out_specs=[
        pl.BlockSpec((_RB, _NCAND), lambda i: (i, 0)),
        pl.BlockSpec((_RB, 1), lambda i: (i, 0)),
    ],
    out_shape=[
        jax.ShapeDtypeStruct((_B, _NCAND), jnp.float32),
        jax.ShapeDtypeStruct((_B, 1), jnp.int32),
    ],
    name="sampler_stage_b_math",
)

def kernel(logits, current_tokens):
    del current_tokens
    _stage_a, _stage_c = _build_sc_stages()
    cand_vals, cand_idx = _stage_a(logits.reshape(_B * _V))
    probs_c, tok = _stage_b(
        cand_vals.reshape(_B, _NCAND), cand_idx.reshape(_B, _NCAND)
    )
    probs = _stage_c(probs_c.reshape(_B * _NCAND), cand_idx)
    return probs.reshape(_B, _V), tok.reshape(_B)


# speedup vs baseline: 48.1264x; 1.1713x over previous
import functools

import jax
import jax.numpy as jnp
from jax import lax
from jax.experimental import pallas as pl
from jax.experimental.pallas import tpu as pltpu
from jax.experimental.pallas import tpu_sc as plsc

_TEMP = 0.8
_K = 50
_P = 0.9
_B = 64
_V = 100000
_NCAND = 256
_CANDBUF = 272
_SLOTS = 64
_NBINS = 4096
_CHUNK = 20000
_NCHUNK = _V // _CHUNK
_L = 16
_VPC = _CHUNK // _L
_NCORES = 2
_NSUB = 16
_UNROLL = 25
_ROWS_PER_W = _B // (_NCORES * _NSUB)

import numpy as np

_MINF = np.float32(-np.inf)
_NEG1E9 = np.float32(-1e9)


def _lane(x16, i):
    return jnp.squeeze(lax.slice_in_dim(x16, i, i + 1, axis=0))


def _monotone_u32(v16):
    s = lax.bitcast_convert_type(v16, jnp.int32)
    m = lax.shift_right_arithmetic(s, 31)
    return s ^ (m | jnp.int32(-(2**31)))


def _stage_a_body(logits_hbm, oval_hbm, oidx_hbm, chunk_v0, chunk_v1, hist, cval, cidx, lbval, lbidx, sem0, sem1):
    wid = lax.axis_index("s") * _NCORES + lax.axis_index("c")
    lane = lax.iota(jnp.int32, _L)
    ones = jnp.ones((_L,), jnp.int32)
    zeros_i = jnp.zeros((_L,), jnp.int32)
    minf16 = jnp.full((_L,), _MINF, jnp.float32)
    neg1_16 = jnp.full((_L,), -1, jnp.int32)

    bufs = (chunk_v0, chunk_v1)
    sems = (sem0, sem1)

    def do_row(r):
        def start(c):
            return pltpu.async_copy(
                logits_hbm.at[pl.ds(r * _V + c * _CHUNK, _CHUNK)], bufs[c % 2], sems[c % 2]
            )

        h0 = start(0)

        def zed(i, _):
            for k in range(16):
                hist[pl.ds((i * 16 + k) * _L, _L)] = zeros_i
            return 0

        lax.fori_loop(0, (_NBINS * _L) // (_L * 16), zed, 0)

        def p1_process(buf):
            def p1_vec(i, _):
                for k in range(_UNROLL):
                    v = buf[pl.ds((i * _UNROLL + k) * _L, _L)]
                    bins = lax.shift_right_logical(_monotone_u32(v), 20)
                    plsc.addupdate_scatter(hist, [lane * _NBINS + bins], ones)
                return 0

            lax.fori_loop(0, _VPC // _UNROLL, p1_vec, 0)

        handles = [h0] + [None] * (_NCHUNK - 1)
        for c in range(_NCHUNK):
            if c + 1 < _NCHUNK:
                handles[c + 1] = start(c + 1)
            handles[c].wait()
            p1_process(bufs[c % 2])

        h2 = start(0)

        def tscan(cc, carry):
            cnt, found, bstar = carry
            q = _NBINS // _L - 1 - cc
            acc = hist[pl.ds(q * _L, _L)]
            for ln in range(1, _L):
                acc = acc + hist[pl.ds(ln * _NBINS + q * _L, _L)]
            cs = plsc.cumsum(lax.rev(acc, (0,)))
            hit = (cs + cnt) >= _K
            j = _lane(plsc.all_reduce_ffs(hit), 0)
            anyhit = j < _L
            cand_b = q * _L + (_L - 1) - j
            first = jnp.logical_and(anyhit, jnp.logical_not(found))
            bstar = jnp.where(first, cand_b, bstar)
            found = jnp.logical_or(found, anyhit)
            cnt = cnt + _lane(cs, _L - 1)
            return cnt, found, bstar

        _, _, bstar = lax.fori_loop(
            0, _NBINS // _L, tscan, (jnp.int32(0), False, jnp.int32(0))
        )

        for i in range(_CANDBUF // _L):
            cval[pl.ds(i * _L, _L)] = minf16
            cidx[pl.ds(i * _L, _L)] = neg1_16

        def p2_process(buf, c, percnt):
            def p2_vec(i, percnt):
                for k in range(_UNROLL):
                    v = buf[pl.ds((i * _UNROLL + k) * _L, _L)]
                    bins = lax.shift_right_logical(_monotone_u32(v), 20)
                    m = bins >= bstar
                    slot = jnp.minimum(percnt, jnp.int32(_SLOTS - 1)) * _L + lane
                    plsc.store_scatter(lbval, [slot], v, mask=m)
                    gidx = c * _CHUNK + (i * _UNROLL + k) * _L + lane
                    plsc.store_scatter(lbidx, [slot], gidx, mask=m)
                    percnt = percnt + m.astype(jnp.int32)
                return percnt

            return lax.fori_loop(0, _VPC // _UNROLL, p2_vec, percnt)

        percnt = jnp.zeros((_L,), jnp.int32)
        handles2 = [h2] + [None] * (_NCHUNK - 1)
        for c in range(_NCHUNK):
            if c + 1 < _NCHUNK:
                handles2[c + 1] = start(c + 1)
            handles2[c].wait()
            percnt = p2_process(bufs[c % 2], c, percnt)

        def compact(sl, off):
            m = percnt > sl
            offc = jnp.minimum(off, jnp.int32(_NCAND))
            plsc.store_compressed(cval.at[pl.ds(offc, _L)], lbval[pl.ds(sl * _L, _L)], mask=m)
            plsc.store_compressed(cidx.at[pl.ds(offc, _L)], lbidx[pl.ds(sl * _L, _L)], mask=m)
            return off + _lane(plsc.all_reduce_population_count(m), 0)

        lax.fori_loop(0, _SLOTS, compact, jnp.int32(0))

        pltpu.sync_copy(cval.at[pl.ds(0, _NCAND)], oval_hbm.at[pl.ds(r * _NCAND, _NCAND)])
        pltpu.sync_copy(cidx.at[pl.ds(0, _NCAND)], oidx_hbm.at[pl.ds(r * _NCAND, _NCAND)])

    for rr in range(_ROWS_PER_W):
        do_row(wid * _ROWS_PER_W + rr)


def _threefry_gumbel(flat_idx):
    k0 = jnp.uint32(0)
    k1 = jnp.uint32(42)
    k2 = jnp.uint32(0x1BD11BDA) ^ k0 ^ k1
    ks = (k0, k1, k2)
    rots = ((13, 15, 26, 6), (17, 29, 16, 24))
    x0 = jnp.zeros(flat_idx.shape, jnp.uint32) + ks[0]
    x1 = flat_idx.astype(jnp.uint32) + ks[1]
    for i in range(5):
        for r in rots[i % 2]:
            x0 = x0 + x1
            x1 = (x1 << jnp.uint32(r)) | (x1 >> jnp.uint32(32 - r))
            x1 = x1 ^ x0
        x0 = x0 + ks[(i + 1) % 3]
        x1 = x1 + ks[(i + 2) % 3] + jnp.uint32(i + 1)
    bits = x0 ^ x1
    f = lax.bitcast_convert_type(
        (bits >> jnp.uint32(9)) | jnp.uint32(0x3F800000), jnp.float32
    ) - jnp.float32(1.0)
    u = f * jnp.float32(1.0 - 1e-10) + jnp.float32(1e-10)
    u = jnp.maximum(jnp.float32(1e-10), u)
    return -jnp.log(-jnp.log(u))


def _stage_b_body(vals_ref, idx_ref, probs_ref, tok_ref):
    pid = pl.program_id(0)
    rb = vals_ref.shape[0]
    v = vals_ref[...]
    ci = idx_ref[...]
    valid = ci >= 0
    lc = v / jnp.float32(_TEMP)
    lc_j = lc[:, None, :]
    lc_i = lc[:, :, None]
    gt = lc_j > lc_i
    cnt = jnp.sum(gt.astype(jnp.float32), axis=-1)
    keep = jnp.logical_and(valid, cnt < _K)
    lm = jnp.where(keep, lc, _NEG1E9)
    m1 = jnp.max(lm, axis=-1, keepdims=True)
    e1 = jnp.exp(lm - m1)
    p1 = e1 / jnp.sum(e1, axis=-1, keepdims=True)
    before = jnp.logical_or(
        gt,
        jnp.logical_and(
            lc_j == lc_i,
            jnp.logical_and(ci[:, None, :] <= ci[:, :, None], valid[:, None, :]),
        ),
    )
    cum = jnp.sum(p1[:, None, :] * before.astype(jnp.float32), axis=-1)
    remove = (cum - p1) > jnp.float32(_P)
    l2 = jnp.where(jnp.logical_and(keep, jnp.logical_not(remove)), lm, _NEG1E9)
    m2 = jnp.max(l2, axis=-1, keepdims=True)
    e2 = jnp.exp(l2 - m2)
    p2 = e2 / jnp.sum(e2, axis=-1, keepdims=True)
    probs_ref[...] = p2

    rows = pid * rb + lax.broadcasted_iota(jnp.int32, (rb, _NCAND), 0)
    flat = rows * _V + jnp.maximum(ci, 0)
    g = _threefry_gumbel(flat)
    score = jnp.log(p2 + jnp.float32(1e-12)) + g
    score = jnp.where(
        jnp.logical_and(valid, p2 > 0), score, jnp.float32(-1e30)
    )
    best = jnp.max(score, axis=-1, keepdims=True)
    tok = jnp.min(jnp.where(score == best, ci, jnp.int32(_V)), axis=-1, keepdims=True)
    tok_ref[...] = tok


def _stage_c_body(pc_hbm, ci_hbm, out_hbm, rowbuf, pval, pidx):
    wid = lax.axis_index("s") * _NCORES + lax.axis_index("c")
    zeros_f = jnp.zeros((_L,), jnp.float32)

    def zed(i, _):
        for k in range(_UNROLL):
            rowbuf[pl.ds((i * _UNROLL + k) * _L, _L)] = zeros_f
        return 0

    lax.fori_loop(0, _V // (_L * _UNROLL), zed, 0)

    def do_row(r):
        pltpu.sync_copy(pc_hbm.at[pl.ds(r * _NCAND, _NCAND)], pval)
        pltpu.sync_copy(ci_hbm.at[pl.ds(r * _NCAND, _NCAND)], pidx)
        for i in range(_NCAND // _L):
            idxv = pidx[pl.ds(i * _L, _L)]
            m = idxv >= 0
            plsc.store_scatter(rowbuf, [idxv], pval[pl.ds(i * _L, _L)], mask=m)
        pltpu.sync_copy(rowbuf, out_hbm.at[pl.ds(r * _V, _V)])
        for i in range(_NCAND // _L):
            idxv = pidx[pl.ds(i * _L, _L)]
            m = idxv >= 0
            plsc.store_scatter(rowbuf, [idxv], zeros_f, mask=m)

    for rr in range(_ROWS_PER_W):
        do_row(wid * _ROWS_PER_W + rr)


@functools.lru_cache(maxsize=1)
def _build_sc_stages():
    mesh = plsc.VectorSubcoreMesh(
        core_axis_name="c", subcore_axis_name="s",
        num_cores=_NCORES, num_subcores=_NSUB,
    )
    stage_a = pl.kernel(
        _stage_a_body,
        out_type=(
            jax.ShapeDtypeStruct((_B * _NCAND,), jnp.float32),
            jax.ShapeDtypeStruct((_B * _NCAND,), jnp.int32),
        ),
        mesh=mesh,
        scratch_types=[
            pltpu.VMEM((_CHUNK,), jnp.float32),
            pltpu.VMEM((_CHUNK,), jnp.float32),
            pltpu.VMEM((_NBINS * _L,), jnp.int32),
            pltpu.VMEM((_CANDBUF,), jnp.float32),
            pltpu.VMEM((_CANDBUF,), jnp.int32),
            pltpu.VMEM((_SLOTS * _L,), jnp.float32),
            pltpu.VMEM((_SLOTS * _L,), jnp.int32),
            pltpu.SemaphoreType.DMA,
            pltpu.SemaphoreType.DMA,
        ],
        name="sampler_stage_a_select",
        compiler_params=pltpu.CompilerParams(needs_layout_passes=False),
    )
    stage_c = pl.kernel(
        _stage_c_body,
        out_type=jax.ShapeDtypeStruct((_B * _V,), jnp.float32),
        mesh=mesh,
        scratch_types=[
            pltpu.VMEM((_V,), jnp.float32),
            pltpu.VMEM((_NCAND,), jnp.float32),
            pltpu.VMEM((_NCAND,), jnp.int32),
        ],
        name="sampler_stage_c_scatter",
        compiler_params=pltpu.CompilerParams(needs_layout_passes=False),
    )
    return stage_a, stage_c


_RB = 8

_stage_b = pl.pallas_call(
    _stage_b_body,
    grid=(_B // _RB,),
    in_specs=[
        pl.BlockSpec((_RB, _NCAND), lambda i: (i, 0)),
        pl.BlockSpec((_RB, _NCAND), lambda i: (i, 0)),
    ],
    out_specs=[
        pl.BlockSpec((_RB, _NCAND), lambda i: (i, 0)),
        pl.BlockSpec((_RB, 1), lambda i: (i, 0)),
    ],
    out_shape=[
        jax.ShapeDtypeStruct((_B, _NCAND), jnp.float32),
        jax.ShapeDtypeStruct((_B, 1), jnp.int32),
    ],
    name="sampler_stage_b_math",
)

def kernel(logits, current_tokens):
    del current_tokens
    _stage_a, _stage_c = _build_sc_stages()
    cand_vals, cand_idx = _stage_a(logits.reshape(_B * _V))
    probs_c, tok = _stage_b(
        cand_vals.reshape(_B, _NCAND), cand_idx.reshape(_B, _NCAND)
    )
    probs = _stage_c(probs_c.reshape(_B * _NCAND), cand_idx)
    return probs.reshape(_B, _V), tok.reshape(_B)


# speedup vs baseline: 48.1919x; 1.0014x over previous
import functools

import jax
import jax.numpy as jnp
from jax import lax
from jax.experimental import pallas as pl
from jax.experimental.pallas import tpu as pltpu
from jax.experimental.pallas import tpu_sc as plsc

_TEMP = 0.8
_K = 50
_P = 0.9
_B = 64
_V = 100000
_NCAND = 256
_CANDBUF = 272
_SLOTS = 64
_NBINS = 4096
_CHUNK = 20000
_NCHUNK = _V // _CHUNK
_L = 16
_VPC = _CHUNK // _L
_NCORES = 2
_NSUB = 16
_UNROLL = 25
_ROWS_PER_W = _B // (_NCORES * _NSUB)

import numpy as np

_MINF = np.float32(-np.inf)
_NEG1E9 = np.float32(-1e9)


def _lane(x16, i):
    return jnp.squeeze(lax.slice_in_dim(x16, i, i + 1, axis=0))


def _monotone_u32(v16):
    s = lax.bitcast_convert_type(v16, jnp.int32)
    m = lax.shift_right_arithmetic(s, 31)
    return s ^ (m | jnp.int32(-(2**31)))


def _stage_a_body(logits_hbm, oval_hbm, oidx_hbm, chunk_v0, chunk_v1, hist, cval, cidx, lbval, lbidx, sem0, sem1):
    wid = lax.axis_index("s") * _NCORES + lax.axis_index("c")
    lane = lax.iota(jnp.int32, _L)
    ones = jnp.ones((_L,), jnp.int32)
    zeros_i = jnp.zeros((_L,), jnp.int32)
    minf16 = jnp.full((_L,), _MINF, jnp.float32)
    neg1_16 = jnp.full((_L,), -1, jnp.int32)

    bufs = (chunk_v0, chunk_v1)
    sems = (sem0, sem1)

    def do_row(r):
        def start(c):
            return pltpu.async_copy(
                logits_hbm.at[pl.ds(r * _V + c * _CHUNK, _CHUNK)], bufs[c % 2], sems[c % 2]
            )

        h0 = start(0)

        def zed(i, _):
            for k in range(16):
                hist[pl.ds((i * 16 + k) * _L, _L)] = zeros_i
            return 0

        lax.fori_loop(0, (_NBINS * _L) // (_L * 16), zed, 0)

        def p1_process(buf):
            def p1_vec(i, _):
                for k in range(_UNROLL):
                    v = buf[pl.ds((i * _UNROLL + k) * _L, _L)]
                    bins = lax.shift_right_logical(_monotone_u32(v), 20)
                    plsc.addupdate_scatter(hist, [lane * _NBINS + bins], ones)
                return 0

            lax.fori_loop(0, _VPC // _UNROLL, p1_vec, 0)

        handles = [h0] + [None] * (_NCHUNK - 1)
        for c in range(_NCHUNK):
            if c + 1 < _NCHUNK:
                handles[c + 1] = start(c + 1)
            handles[c].wait()
            p1_process(bufs[c % 2])

        h2 = start(0)

        def t_cond(carry):
            q, cnt, _ = carry
            return jnp.logical_and(cnt < _K, q >= 0)

        def t_step(carry):
            q, cnt, bstar = carry
            acc = hist[pl.ds(q * _L, _L)]
            for ln in range(1, _L):
                acc = acc + hist[pl.ds(ln * _NBINS + q * _L, _L)]
            cs = plsc.cumsum(lax.rev(acc, (0,)))
            hit = (cs + cnt) >= _K
            j = _lane(plsc.all_reduce_ffs(hit), 0)
            anyhit = j < _L
            cand_b = q * _L + (_L - 1) - jnp.minimum(j, jnp.int32(_L - 1))
            bstar = jnp.where(anyhit, cand_b, bstar)
            cnt = cnt + _lane(cs, _L - 1)
            return q - 1, cnt, bstar

        _, _, bstar = lax.while_loop(
            t_cond, t_step, (jnp.int32(_NBINS // _L - 1), jnp.int32(0), jnp.int32(0))
        )

        for i in range(_CANDBUF // _L):
            cval[pl.ds(i * _L, _L)] = minf16
            cidx[pl.ds(i * _L, _L)] = neg1_16

        def p2_process(buf, c, percnt):
            def p2_vec(i, percnt):
                for k in range(_UNROLL):
                    v = buf[pl.ds((i * _UNROLL + k) * _L, _L)]
                    bins = lax.shift_right_logical(_monotone_u32(v), 20)
                    m = bins >= bstar
                    slot = jnp.minimum(percnt, jnp.int32(_SLOTS - 1)) * _L + lane
                    plsc.store_scatter(lbval, [slot], v, mask=m)
                    gidx = c * _CHUNK + (i * _UNROLL + k) * _L + lane
                    plsc.store_scatter(lbidx, [slot], gidx, mask=m)
                    percnt = percnt + m.astype(jnp.int32)
                return percnt

            return lax.fori_loop(0, _VPC // _UNROLL, p2_vec, percnt)

        percnt = jnp.zeros((_L,), jnp.int32)
        handles2 = [h2] + [None] * (_NCHUNK - 1)
        for c in range(_NCHUNK):
            if c + 1 < _NCHUNK:
                handles2[c + 1] = start(c + 1)
            handles2[c].wait()
            percnt = p2_process(bufs[c % 2], c, percnt)

        def compact(sl, off):
            m = percnt > sl
            offc = jnp.minimum(off, jnp.int32(_NCAND))
            plsc.store_compressed(cval.at[pl.ds(offc, _L)], lbval[pl.ds(sl * _L, _L)], mask=m)
            plsc.store_compressed(cidx.at[pl.ds(offc, _L)], lbidx[pl.ds(sl * _L, _L)], mask=m)
            return off + _lane(plsc.all_reduce_population_count(m), 0)

        lax.fori_loop(0, _SLOTS, compact, jnp.int32(0))

        pltpu.sync_copy(cval.at[pl.ds(0, _NCAND)], oval_hbm.at[pl.ds(r * _NCAND, _NCAND)])
        pltpu.sync_copy(cidx.at[pl.ds(0, _NCAND)], oidx_hbm.at[pl.ds(r * _NCAND, _NCAND)])

    for rr in range(_ROWS_PER_W):
        do_row(wid * _ROWS_PER_W + rr)


def _threefry_gumbel(flat_idx):
    k0 = jnp.uint32(0)
    k1 = jnp.uint32(42)
    k2 = jnp.uint32(0x1BD11BDA) ^ k0 ^ k1
    ks = (k0, k1, k2)
    rots = ((13, 15, 26, 6), (17, 29, 16, 24))
    x0 = jnp.zeros(flat_idx.shape, jnp.uint32) + ks[0]
    x1 = flat_idx.astype(jnp.uint32) + ks[1]
    for i in range(5):
        for r in rots[i % 2]:
            x0 = x0 + x1
            x1 = (x1 << jnp.uint32(r)) | (x1 >> jnp.uint32(32 - r))
            x1 = x1 ^ x0
        x0 = x0 + ks[(i + 1) % 3]
        x1 = x1 + ks[(i + 2) % 3] + jnp.uint32(i + 1)
    bits = x0 ^ x1
    f = lax.bitcast_convert_type(
        (bits >> jnp.uint32(9)) | jnp.uint32(0x3F800000), jnp.float32
    ) - jnp.float32(1.0)
    u = f * jnp.float32(1.0 - 1e-10) + jnp.float32(1e-10)
    u = jnp.maximum(jnp.float32(1e-10), u)
    return -jnp.log(-jnp.log(u))


def _stage_b_body(vals_ref, idx_ref, probs_ref, tok_ref):
    pid = pl.program_id(0)
    rb = vals_ref.shape[0]
    v = vals_ref[...]
    ci = idx_ref[...]
    valid = ci >= 0
    lc = v / jnp.float32(_TEMP)
    lc_j = lc[:, None, :]
    lc_i = lc[:, :, None]
    gt = lc_j > lc_i
    cnt = jnp.sum(gt.astype(jnp.float32), axis=-1)
    keep = jnp.logical_and(valid, cnt < _K)
    lm = jnp.where(keep, lc, _NEG1E9)
    m1 = jnp.max(lm, axis=-1, keepdims=True)
    e1 = jnp.exp(lm - m1)
    p1 = e1 / jnp.sum(e1, axis=-1, keepdims=True)
    before = jnp.logical_or(
        gt,
        jnp.logical_and(
            lc_j == lc_i,
            jnp.logical_and(ci[:, None, :] <= ci[:, :, None], valid[:, None, :]),
        ),
    )
    cum = jnp.sum(p1[:, None, :] * before.astype(jnp.float32), axis=-1)
    remove = (cum - p1) > jnp.float32(_P)
    l2 = jnp.where(jnp.logical_and(keep, jnp.logical_not(remove)), lm, _NEG1E9)
    m2 = jnp.max(l2, axis=-1, keepdims=True)
    e2 = jnp.exp(l2 - m2)
    p2 = e2 / jnp.sum(e2, axis=-1, keepdims=True)
    probs_ref[...] = p2

    rows = pid * rb + lax.broadcasted_iota(jnp.int32, (rb, _NCAND), 0)
    flat = rows * _V + jnp.maximum(ci, 0)
    g = _threefry_gumbel(flat)
    score = jnp.log(p2 + jnp.float32(1e-12)) + g
    score = jnp.where(
        jnp.logical_and(valid, p2 > 0), score, jnp.float32(-1e30)
    )
    best = jnp.max(score, axis=-1, keepdims=True)
    tok = jnp.min(jnp.where(score == best, ci, jnp.int32(_V)), axis=-1, keepdims=True)
    tok_ref[...] = tok


def _stage_c_body(pc_hbm, ci_hbm, out_hbm, rowbuf, pval, pidx):
    wid = lax.axis_index("s") * _NCORES + lax.axis_index("c")
    zeros_f = jnp.zeros((_L,), jnp.float32)

    def zed(i, _):
        for k in range(_UNROLL):
            rowbuf[pl.ds((i * _UNROLL + k) * _L, _L)] = zeros_f
        return 0

    lax.fori_loop(0, _V // (_L * _UNROLL), zed, 0)

    def do_row(r):
        pltpu.sync_copy(pc_hbm.at[pl.ds(r * _NCAND, _NCAND)], pval)
        pltpu.sync_copy(ci_hbm.at[pl.ds(r * _NCAND, _NCAND)], pidx)
        for i in range(_NCAND // _L):
            idxv = pidx[pl.ds(i * _L, _L)]
            m = idxv >= 0
            plsc.store_scatter(rowbuf, [idxv], pval[pl.ds(i * _L, _L)], mask=m)
        pltpu.sync_copy(rowbuf, out_hbm.at[pl.ds(r * _V, _V)])
        for i in range(_NCAND // _L):
            idxv = pidx[pl.ds(i * _L, _L)]
            m = idxv >= 0
            plsc.store_scatter(rowbuf, [idxv], zeros_f, mask=m)

    for rr in range(_ROWS_PER_W):
        do_row(wid * _ROWS_PER_W + rr)


@functools.lru_cache(maxsize=1)
def _build_sc_stages():
    mesh = plsc.VectorSubcoreMesh(
        core_axis_name="c", subcore_axis_name="s",
        num_cores=_NCORES, num_subcores=_NSUB,
    )
    stage_a = pl.kernel(
        _stage_a_body,
        out_type=(
            jax.ShapeDtypeStruct((_B * _NCAND,), jnp.float32),
            jax.ShapeDtypeStruct((_B * _NCAND,), jnp.int32),
        ),
        mesh=mesh,
        scratch_types=[
            pltpu.VMEM((_CHUNK,), jnp.float32),
            pltpu.VMEM((_CHUNK,), jnp.float32),
            pltpu.VMEM((_NBINS * _L,), jnp.int32),
            pltpu.VMEM((_CANDBUF,), jnp.float32),
            pltpu.VMEM((_CANDBUF,), jnp.int32),
            pltpu.VMEM((_SLOTS * _L,), jnp.float32),
            pltpu.VMEM((_SLOTS * _L,), jnp.int32),
            pltpu.SemaphoreType.DMA,
            pltpu.SemaphoreType.DMA,
        ],
        name="sampler_stage_a_select",
        compiler_params=pltpu.CompilerParams(needs_layout_passes=False),
    )
    stage_c = pl.kernel(
        _stage_c_body,
        out_type=jax.ShapeDtypeStruct((_B * _V,), jnp.float32),
        mesh=mesh,
        scratch_types=[
            pltpu.VMEM((_V,), jnp.float32),
            pltpu.VMEM((_NCAND,), jnp.float32),
            pltpu.VMEM((_NCAND,), jnp.int32),
        ],
        name="sampler_stage_c_scatter",
        compiler_params=pltpu.CompilerParams(needs_layout_passes=False),
    )
    return stage_a, stage_c


_RB = 8

_stage_b = pl.pallas_call(
    _stage_b_body,
    grid=(_B // _RB,),
    in_specs=[
        pl.BlockSpec((_RB, _NCAND), lambda i: (i, 0)),
        pl.BlockSpec((_RB, _NCAND), lambda i: (i, 0)),
    ],
    out_specs=[
        pl.BlockSpec((_RB, _NCAND), lambda i: (i, 0)),
        pl.BlockSpec((_RB, 1), lambda i: (i, 0)),
    ],
    out_shape=[
        jax.ShapeDtypeStruct((_B, _NCAND), jnp.float32),
        jax.ShapeDtypeStruct((_B, 1), jnp.int32),
    ],
    name="sampler_stage_b_math",
)

def kernel(logits, current_tokens):
    del current_tokens
    _stage_a, _stage_c = _build_sc_stages()
    cand_vals, cand_idx = _stage_a(logits.reshape(_B * _V))
    probs_c, tok = _stage_b(
        cand_vals.reshape(_B, _NCAND), cand_idx.reshape(_B, _NCAND)
    )
    probs = _stage_c(probs_c.reshape(_B * _NCAND), cand_idx)
    return probs.reshape(_B, _V), tok.reshape(_B)
